# baseline (speedup 1.0000x reference)
import functools

import jax
import jax.numpy as jnp
from jax import lax
from jax.experimental import pallas as pl
from jax.experimental.pallas import tpu as pltpu
from jax.experimental.pallas import tpu_sc as plsc

_VOCAB_TILE = 5632


def _matmul_body(emb_ref, w_ref, b_ref, out_ref):
    acc = lax.dot_general(
        emb_ref[...], w_ref[...],
        (((1,), (1,)), ((), ())),
        preferred_element_type=jnp.float32,
    )
    out_ref[...] = acc + b_ref[...]


def _make_sc_gather(V, H, T):
    info = plsc.get_sparse_core_info()
    num_workers = 1 * info.num_subcores
    rows_per_worker = T // num_workers
    mesh = plsc.VectorSubcoreMesh(
        core_axis_name="c", subcore_axis_name="s", num_cores=1
    )

    @functools.partial(
        pl.kernel,
        mesh=mesh,
        out_type=jax.ShapeDtypeStruct((T, H), jnp.float32),
        scratch_types=[
            pltpu.VMEM((rows_per_worker,), jnp.int32),
            pltpu.VMEM((rows_per_worker, H), jnp.float32),
            pltpu.SemaphoreType.DMA,
        ],
    )
    def gather(table_hbm, idx_hbm, out_hbm, idx_v, rows_v, sem):
        wid = lax.axis_index("s") * 1 + lax.axis_index("c")
        pltpu.sync_copy(idx_hbm.at[wid], idx_v)
        pltpu.async_copy(table_hbm.at[idx_v], rows_v, sem).wait()
        pltpu.sync_copy(rows_v, out_hbm.at[pl.ds(wid * rows_per_worker, rows_per_worker)])

    return lambda table, ids: gather(
        table, ids.reshape(num_workers, rows_per_worker)
    )


def kernel(input_ids, embedding, lm_head_w, lm_head_b):
    B, L = input_ids.shape
    V, H = embedding.shape
    T = B * L
    embeds = _make_sc_gather(V, H, T)(embedding, input_ids.astype(jnp.int32))

    nv = pl.cdiv(V, _VOCAB_TILE)
    logits = pl.pallas_call(
        _matmul_body,
        grid=(nv,),
        in_specs=[
            pl.BlockSpec((T, H), lambda j: (0, 0)),
            pl.BlockSpec((_VOCAB_TILE, H), lambda j: (j, 0)),
            pl.BlockSpec((1, _VOCAB_TILE), lambda j: (0, j)),
        ],
        out_specs=pl.BlockSpec((T, _VOCAB_TILE), lambda j: (0, j)),
        out_shape=jax.ShapeDtypeStruct((T, V), jnp.float32),
    )(embeds, lm_head_w, lm_head_b.reshape(1, V))

    return logits.reshape(B, L, V)


# speedup vs baseline: 1.0024x; 1.0024x over previous
import functools

import jax
import jax.numpy as jnp
from jax import lax
from jax.experimental import pallas as pl
from jax.experimental.pallas import tpu as pltpu
from jax.experimental.pallas import tpu_sc as plsc

_VOCAB_TILE = 4608


def _matmul_body(emb_ref, w_ref, b_ref, out_ref):
    acc = lax.dot_general(
        emb_ref[...], w_ref[...],
        (((1,), (1,)), ((), ())),
        preferred_element_type=jnp.float32,
    )
    out_ref[...] = acc + b_ref[...]


def _make_sc_gather(V, H, T):
    info = plsc.get_sparse_core_info()
    num_workers = 1 * info.num_subcores
    rows_per_worker = T // num_workers
    mesh = plsc.VectorSubcoreMesh(
        core_axis_name="c", subcore_axis_name="s", num_cores=1
    )

    @functools.partial(
        pl.kernel,
        mesh=mesh,
        out_type=jax.ShapeDtypeStruct((T, H), jnp.float32),
        scratch_types=[
            pltpu.VMEM((rows_per_worker,), jnp.int32),
            pltpu.VMEM((rows_per_worker, H), jnp.float32),
            pltpu.SemaphoreType.DMA,
        ],
    )
    def gather(table_hbm, idx_hbm, out_hbm, idx_v, rows_v, sem):
        wid = lax.axis_index("s") * 1 + lax.axis_index("c")
        pltpu.sync_copy(idx_hbm.at[wid], idx_v)
        pltpu.async_copy(table_hbm.at[idx_v], rows_v, sem).wait()
        pltpu.sync_copy(rows_v, out_hbm.at[pl.ds(wid * rows_per_worker, rows_per_worker)])

    return lambda table, ids: gather(
        table, ids.reshape(num_workers, rows_per_worker)
    )


def kernel(input_ids, embedding, lm_head_w, lm_head_b):
    B, L = input_ids.shape
    V, H = embedding.shape
    T = B * L
    embeds = _make_sc_gather(V, H, T)(embedding, input_ids.astype(jnp.int32))

    nv = pl.cdiv(V, _VOCAB_TILE)
    logits = pl.pallas_call(
        _matmul_body,
        grid=(nv,),
        in_specs=[
            pl.BlockSpec((T, H), lambda j: (0, 0)),
            pl.BlockSpec((_VOCAB_TILE, H), lambda j: (j, 0)),
            pl.BlockSpec((1, _VOCAB_TILE), lambda j: (0, j)),
        ],
        out_specs=pl.BlockSpec((T, _VOCAB_TILE), lambda j: (0, j)),
        out_shape=jax.ShapeDtypeStruct((T, V), jnp.float32),
    )(embeds, lm_head_w, lm_head_b.reshape(1, V))

    return logits.reshape(B, L, V)


# speedup vs baseline: 1.0042x; 1.0017x over previous
import functools

import jax
import jax.numpy as jnp
from jax import lax
from jax.experimental import pallas as pl
from jax.experimental.pallas import tpu as pltpu
from jax.experimental.pallas import tpu_sc as plsc

_VOCAB_TILE = 5120


def _matmul_body(emb_ref, w_ref, b_ref, out_ref):
    acc = lax.dot_general(
        emb_ref[...], w_ref[...],
        (((1,), (1,)), ((), ())),
        preferred_element_type=jnp.float32,
    )
    out_ref[...] = acc + b_ref[...]


def _make_sc_gather(V, H, T):
    info = plsc.get_sparse_core_info()
    num_workers = info.num_subcores
    rows_per_worker = T // num_workers
    mesh = plsc.VectorSubcoreMesh(
        core_axis_name="c", subcore_axis_name="s", num_cores=1
    )

    @functools.partial(
        pl.kernel,
        mesh=mesh,
        out_type=jax.ShapeDtypeStruct((T, H), jnp.float32),
        scratch_types=[
            pltpu.VMEM((rows_per_worker,), jnp.int32),
            pltpu.VMEM((rows_per_worker, H), jnp.float32),
            pltpu.SemaphoreType.DMA,
        ],
    )
    def gather(table_hbm, idx_hbm, out_hbm, idx_v, rows_v, sem):
        wid = lax.axis_index("s")
        pltpu.sync_copy(idx_hbm.at[wid], idx_v)
        pltpu.async_copy(table_hbm.at[idx_v], rows_v, sem).wait()
        pltpu.sync_copy(
            rows_v, out_hbm.at[pl.ds(wid * rows_per_worker, rows_per_worker)]
        )

    return lambda table, ids: gather(
        table, ids.reshape(num_workers, rows_per_worker)
    )


def kernel(input_ids, embedding, lm_head_w, lm_head_b):
    B, L = input_ids.shape
    V, H = embedding.shape
    T = B * L
    embeds = _make_sc_gather(V, H, T)(embedding, input_ids.astype(jnp.int32))

    nv = pl.cdiv(V, _VOCAB_TILE)
    logits = pl.pallas_call(
        _matmul_body,
        grid=(nv,),
        in_specs=[
            pl.BlockSpec((T, H), lambda j: (0, 0)),
            pl.BlockSpec((_VOCAB_TILE, H), lambda j: (j, 0)),
            pl.BlockSpec((1, _VOCAB_TILE), lambda j: (0, j)),
        ],
        out_specs=pl.BlockSpec((T, _VOCAB_TILE), lambda j: (0, j)),
        out_shape=jax.ShapeDtypeStruct((T, V), jnp.float32),
    )(embeds, lm_head_w, lm_head_b.reshape(1, V))

    return logits.reshape(B, L, V)
